# baseline (speedup 1.0000x reference)
import functools

import jax
import jax.numpy as jnp
from jax.experimental import pallas as pl
from jax.experimental.pallas import tpu as pltpu

K_TOP = 2
LOOKAHEAD = 2


def _tile_partial_sum(t):
    tl, d = t.shape
    return jnp.sum(t.reshape(tl // 8, 8, d), axis=0)


def _router_weights(pooled, wrg):
    logits = jax.lax.dot_general(pooled, wrg, (((1,), (1,)), ((), ())),
                                 precision=jax.lax.Precision.HIGHEST,
                                 preferred_element_type=jnp.float32)
    m = jnp.max(logits, axis=-1, keepdims=True)
    p = jnp.exp(logits - m)
    probs = p / jnp.sum(p, axis=-1, keepdims=True)
    e_ids = jax.lax.broadcasted_iota(jnp.int32, probs.shape, 1)
    v1 = jnp.max(probs, axis=-1, keepdims=True)
    i1 = jnp.argmax(probs, axis=-1)[:, None]
    masked = jnp.where(e_ids == i1, -jnp.inf, probs)
    v2 = jnp.max(masked, axis=-1, keepdims=True)
    i2 = jnp.argmax(masked, axis=-1)[:, None]
    denom = v1 + v2
    w = jnp.where(e_ids == i1, v1 / denom, 0.0)
    w = jnp.where(e_ids == i2, v2 / denom, w)
    return w


def _fused_kernel(xh_ref, wg_ref, wr_ref, a_ref, bcat_ref, wbt_ref, out_ref,
                  stage, xbuf, acc, awbuf, wrg, sems, *, nb, nlt, tl, r,
                  inv_l):
    bb = pl.program_id(0)
    lt = pl.program_id(1)
    ns = LOOKAHEAD + 1
    g = bb * nlt + lt
    n_tiles = nb * nlt

    def tile_copy(gt):
        slot = jax.lax.rem(gt, ns)
        return pltpu.make_async_copy(
            xh_ref.at[gt // nlt, pl.ds(jax.lax.rem(gt, nlt) * tl, tl), :],
            stage.at[slot],
            sems.at[slot],
        )

    @pl.when(g == 0)
    def _():
        for q in range(LOOKAHEAD):
            tile_copy(q).start()
        d = wrg.shape[1]
        wg_copy = pltpu.make_async_copy(
            wg_ref, stage.at[LOOKAHEAD, pl.ds(0, d), :], sems.at[LOOKAHEAD])
        wg_copy.start()
        wg_copy.wait()
        wrg[...] = jax.lax.dot_general(
            wr_ref[...], stage[LOOKAHEAD, pl.ds(0, d), :],
            (((1,), (0,)), ((), ())),
            precision=jax.lax.Precision.HIGHEST,
            preferred_element_type=jnp.float32)

    @pl.when(g + LOOKAHEAD < n_tiles)
    def _():
        tile_copy(g + LOOKAHEAD).start()

    par_c = jax.lax.rem(bb + 1, 2)

    @pl.when(bb >= 1)
    def _():
        @pl.when(lt == 0)
        def _():
            pooled = jnp.sum(acc[par_c], axis=0, keepdims=True) * inv_l
            w = _router_weights(pooled, wrg[...])
            e = w.shape[1]
            k_exp = jax.lax.broadcasted_iota(jnp.int32, (e * r, e), 0) // r
            eids = jax.lax.broadcasted_iota(jnp.int32, (e * r, e), 1)
            sel = (k_exp == eids).astype(jnp.float32)
            w_rep = jnp.sum(sel * w, axis=1, keepdims=True)
            awbuf[...] = (a_ref[...] * w_rep).astype(awbuf.dtype)

        xt = xbuf[par_c, pl.ds(lt * tl, tl), :]
        z = jax.lax.dot_general(
            xt, awbuf[...], (((1,), (1,)), ((), ())),
            preferred_element_type=jnp.float32)
        base = jnp.dot(xt, wbt_ref[...],
                       preferred_element_type=jnp.float32)
        lora = jax.lax.dot_general(
            z.astype(jnp.bfloat16), bcat_ref[...], (((1,), (0,)), ((), ())),
            preferred_element_type=jnp.float32)
        out_ref[0] = base + lora

    par_p = jax.lax.rem(bb, 2)

    @pl.when(g < n_tiles)
    def _():
        tile_copy(g).wait()
        t = stage[jax.lax.rem(g, ns)]
        xbuf[par_p, pl.ds(lt * tl, tl), :] = t.astype(jnp.bfloat16)
        part = _tile_partial_sum(t)

        @pl.when(lt == 0)
        def _():
            acc[par_p] = part

        @pl.when(lt > 0)
        def _():
            acc[par_p] += part


@jax.jit
def kernel(x, W_base, W_g, W_r, lora_A, lora_B):
    B, L, D = x.shape
    E, R, _ = lora_A.shape

    A_cat = lora_A.reshape(E * R, D)
    B_cat = lora_B.transpose(0, 2, 1).reshape(E * R, D).astype(jnp.bfloat16)
    Wb_t = W_base.T.astype(jnp.bfloat16)

    TL = 2048
    NLT = L // TL

    out = pl.pallas_call(
        functools.partial(_fused_kernel, nb=B, nlt=NLT, tl=TL, r=R,
                          inv_l=1.0 / L),
        grid=(B + 1, NLT),
        in_specs=[
            pl.BlockSpec(memory_space=pltpu.MemorySpace.HBM),
            pl.BlockSpec(memory_space=pltpu.MemorySpace.HBM),
            pl.BlockSpec((E, D), lambda bb, lt: (0, 0)),
            pl.BlockSpec((E * R, D), lambda bb, lt: (0, 0)),
            pl.BlockSpec((E * R, D), lambda bb, lt: (0, 0)),
            pl.BlockSpec((D, D), lambda bb, lt: (0, 0)),
        ],
        out_specs=pl.BlockSpec(
            (1, TL, D),
            lambda bb, lt: (jnp.maximum(bb - 1, 0),
                            jnp.where(bb == 0, 0, lt), 0)),
        out_shape=jax.ShapeDtypeStruct((B, L, D), jnp.float32),
        scratch_shapes=[
            pltpu.VMEM((LOOKAHEAD + 1, TL, D), jnp.float32),
            pltpu.VMEM((2, L, D), jnp.bfloat16),
            pltpu.VMEM((2, 8, D), jnp.float32),
            pltpu.VMEM((E * R, D), jnp.bfloat16),
            pltpu.VMEM((E, D), jnp.float32),
            pltpu.SemaphoreType.DMA((LOOKAHEAD + 1,)),
        ],
        compiler_params=pltpu.CompilerParams(
            vmem_limit_bytes=63 * 1024 * 1024),
    )(x, W_g, W_r, A_cat, B_cat, Wb_t)

    return out


# speedup vs baseline: 1.5480x; 1.5480x over previous
import functools

import jax
import jax.numpy as jnp
from jax.experimental import pallas as pl
from jax.experimental.pallas import tpu as pltpu

K_TOP = 2
LOOKAHEAD = 2


def _tile_partial_sum(t):
    tl, d = t.shape
    return jnp.sum(t.reshape(tl // 8, 8, d), axis=0)


def _router_weights(pooled, wrg):
    logits = jax.lax.dot_general(pooled, wrg, (((1,), (1,)), ((), ())),
                                 precision=jax.lax.Precision.HIGHEST,
                                 preferred_element_type=jnp.float32)
    m = jnp.max(logits, axis=-1, keepdims=True)
    p = jnp.exp(logits - m)
    probs = p / jnp.sum(p, axis=-1, keepdims=True)
    e_ids = jax.lax.broadcasted_iota(jnp.int32, probs.shape, 1)
    v1 = jnp.max(probs, axis=-1, keepdims=True)
    i1 = jnp.argmax(probs, axis=-1)[:, None]
    masked = jnp.where(e_ids == i1, -jnp.inf, probs)
    v2 = jnp.max(masked, axis=-1, keepdims=True)
    i2 = jnp.argmax(masked, axis=-1)[:, None]
    denom = v1 + v2
    w = jnp.where(e_ids == i1, v1 / denom, 0.0)
    w = jnp.where(e_ids == i2, v2 / denom, w)
    return w


def _fused_kernel(xh_ref, wg_ref, wr_ref, a_ref, bcat_ref, wbt_ref, out_ref,
                  stage, xbuf, acc, mt, wrg, sems, *, nb, nlt, tl, r, inv_l):
    bb = pl.program_id(0)
    lt = pl.program_id(1)
    ns = LOOKAHEAD + 1
    g = bb * nlt + lt
    n_tiles = nb * nlt

    def tile_copy(gt):
        slot = jax.lax.rem(gt, ns)
        return pltpu.make_async_copy(
            xh_ref.at[gt // nlt, pl.ds(jax.lax.rem(gt, nlt) * tl, tl), :],
            stage.at[slot],
            sems.at[slot],
        )

    @pl.when(g == 0)
    def _():
        for q in range(LOOKAHEAD):
            tile_copy(q).start()
        d = wrg.shape[1]
        wg_copy = pltpu.make_async_copy(
            wg_ref, stage.at[LOOKAHEAD, pl.ds(0, d), :], sems.at[LOOKAHEAD])
        wg_copy.start()
        wg_copy.wait()
        wrg[...] = jax.lax.dot_general(
            wr_ref[...], stage[LOOKAHEAD, pl.ds(0, d), :],
            (((1,), (0,)), ((), ())),
            precision=jax.lax.Precision.HIGHEST,
            preferred_element_type=jnp.float32)

    @pl.when(g + LOOKAHEAD < n_tiles)
    def _():
        tile_copy(g + LOOKAHEAD).start()

    par_c = jax.lax.rem(bb + 1, 2)

    @pl.when(bb >= 1)
    def _():
        @pl.when(lt == 0)
        def _():
            pooled = jnp.sum(acc[par_c], axis=0, keepdims=True) * inv_l
            w = _router_weights(pooled, wrg[...])
            e = w.shape[1]
            k_exp = jax.lax.broadcasted_iota(jnp.int32, (e * r, e), 0) // r
            eids = jax.lax.broadcasted_iota(jnp.int32, (e * r, e), 1)
            sel = (k_exp == eids).astype(jnp.float32)
            w_rep = jnp.sum(sel * w, axis=1, keepdims=True)
            a_w = a_ref[...] * w_rep
            delta = jax.lax.dot_general(
                a_w, bcat_ref[...], (((0,), (0,)), ((), ())),
                preferred_element_type=jnp.float32)
            mt[...] = (wbt_ref[...] + delta).astype(mt.dtype)

        xt = xbuf[par_c, pl.ds(lt * tl, tl), :]
        out_ref[0] = jnp.dot(xt, mt[...],
                             preferred_element_type=jnp.float32)

    par_p = jax.lax.rem(bb, 2)

    @pl.when(g < n_tiles)
    def _():
        tile_copy(g).wait()
        t = stage[jax.lax.rem(g, ns)]
        xbuf[par_p, pl.ds(lt * tl, tl), :] = t.astype(jnp.bfloat16)
        part = _tile_partial_sum(t)

        @pl.when(lt == 0)
        def _():
            acc[par_p] = part

        @pl.when(lt > 0)
        def _():
            acc[par_p] += part


@jax.jit
def kernel(x, W_base, W_g, W_r, lora_A, lora_B):
    B, L, D = x.shape
    E, R, _ = lora_A.shape

    A_cat = lora_A.reshape(E * R, D)
    B_cat = lora_B.transpose(0, 2, 1).reshape(E * R, D)
    Wb_t = W_base.T

    TL = 2048
    NLT = L // TL

    out = pl.pallas_call(
        functools.partial(_fused_kernel, nb=B, nlt=NLT, tl=TL, r=R,
                          inv_l=1.0 / L),
        grid=(B + 1, NLT),
        in_specs=[
            pl.BlockSpec(memory_space=pltpu.MemorySpace.HBM),
            pl.BlockSpec(memory_space=pltpu.MemorySpace.HBM),
            pl.BlockSpec((E, D), lambda bb, lt: (0, 0)),
            pl.BlockSpec((E * R, D), lambda bb, lt: (0, 0)),
            pl.BlockSpec((E * R, D), lambda bb, lt: (0, 0)),
            pl.BlockSpec((D, D), lambda bb, lt: (0, 0)),
        ],
        out_specs=pl.BlockSpec(
            (1, TL, D),
            lambda bb, lt: (jnp.maximum(bb - 1, 0),
                            jnp.where(bb == 0, 0, lt), 0)),
        out_shape=jax.ShapeDtypeStruct((B, L, D), jnp.float32),
        scratch_shapes=[
            pltpu.VMEM((LOOKAHEAD + 1, TL, D), jnp.float32),
            pltpu.VMEM((2, L, D), jnp.bfloat16),
            pltpu.VMEM((2, 8, D), jnp.float32),
            pltpu.VMEM((D, D), jnp.bfloat16),
            pltpu.VMEM((E, D), jnp.float32),
            pltpu.SemaphoreType.DMA((LOOKAHEAD + 1,)),
        ],
        compiler_params=pltpu.CompilerParams(
            vmem_limit_bytes=63 * 1024 * 1024),
    )(x, W_g, W_r, A_cat, B_cat, Wb_t)

    return out
